# baseline (speedup 1.0000x reference)
import jax
import jax.numpy as jnp
from jax.experimental import pallas as pl
from jax.experimental.pallas import tpu as pltpu

_B, _S, _I = 4, 2048, 1024
_E, _R = 8, 128
_H = _E * _R
_CHUNK = 256
_NCH = _S // _CHUNK
_TB = 512
_NTB = (_B * _S) // _TB


def _gru_body(xT_ref, wih_ref, whh_ref, h0_ref, ys_ref, hT_ref, h_scr, gi_scr):
    i = pl.program_id(0)

    @pl.when(i == 0)
    def _init():
        h_scr[...] = h0_ref[...]

    x2 = xT_ref[...].reshape(_CHUNK * _B, _I)
    gi_scr[...] = jnp.dot(x2, wih_ref[...])

    def step2(u, h):
        gi8 = gi_scr[pl.ds(u * 2 * _B, 2 * _B), :]
        for k in range(2):
            gi_t = gi8[k * _B:(k + 1) * _B, :]
            gh_rz = jnp.dot(h, whh_ref[:, :2 * _H])
            gh_n = jnp.dot(h, whh_ref[:, 2 * _H:])
            r = jax.nn.sigmoid(gi_t[:, :_H] + gh_rz[:, :_H])
            z = jax.nn.sigmoid(gi_t[:, _H:2 * _H] + gh_rz[:, _H:])
            n = jnp.tanh(gi_t[:, 2 * _H:] + r * gh_n)
            h = (1.0 - z) * n + z * h
            ys_ref[u * 2 + k] = h
        return h

    h_fin = jax.lax.fori_loop(0, _CHUNK // 2, step2, h_scr[...])
    h_scr[...] = h_fin
    hT_ref[...] = h_fin


def _gru_call(xT, wih_t, whh_t, h0):
    return pl.pallas_call(
        _gru_body,
        grid=(_NCH,),
        in_specs=[
            pl.BlockSpec((_CHUNK, _B, _I), lambda i: (i, 0, 0)),
            pl.BlockSpec((_I, 3 * _H), lambda i: (0, 0)),
            pl.BlockSpec((_H, 3 * _H), lambda i: (0, 0)),
            pl.BlockSpec((_B, _H), lambda i: (0, 0)),
        ],
        out_specs=[
            pl.BlockSpec((_CHUNK, _B, _H), lambda i: (i, 0, 0)),
            pl.BlockSpec((_B, _H), lambda i: (0, 0)),
        ],
        out_shape=[
            jax.ShapeDtypeStruct((_S, _B, _H), jnp.float32),
            jax.ShapeDtypeStruct((_B, _H), jnp.float32),
        ],
        scratch_shapes=[
            pltpu.VMEM((_B, _H), jnp.float32),
            pltpu.VMEM((_CHUNK * _B, 3 * _H), jnp.float32),
        ],
    )(xT, wih_t, whh_t, h0)


def _tail_body(v_ref, rn_ref, cs_ref, pen_ref, acc_ref):
    j = pl.program_id(0)
    v = v_ref[...]
    n = jnp.sqrt(jnp.sum(v * v, axis=-1, keepdims=True))
    rn = v / jnp.maximum(n, 1e-12)
    rn_ref[...] = rn
    s = jnp.sum(rn * rn, axis=-1)
    na = jnp.maximum(jnp.sqrt(s), 1e-8)
    cs_ref[...] = 1.0 - s / (na * na)

    dsq = {}
    for a in range(_E):
        for b in range(a, _E):
            g = jnp.sum(rn[:, a, :] * rn[:, b, :], axis=-1, keepdims=True)
            d = g - (1.0 if a == b else 0.0)
            dsq[(a, b)] = d * d
    pen_tok = jnp.zeros((_TB, 1), jnp.float32)
    for a in range(_E):
        rowsq = jnp.zeros((_TB, 1), jnp.float32)
        for b in range(_E):
            rowsq = rowsq + dsq[(min(a, b), max(a, b))]
        m = jnp.maximum(jnp.sqrt(rowsq), 1e-12)
        pen_tok = pen_tok + rowsq / (m * m)
    blk = jnp.sum(pen_tok)

    @pl.when(j == 0)
    def _first():
        acc_ref[0] = blk

    @pl.when(j > 0)
    def _rest():
        acc_ref[0] = acc_ref[0] + blk

    @pl.when(j == _NTB - 1)
    def _last():
        pen_ref[0, 0] = acc_ref[0] / float(_B * _S)


def _tail_call(routing):
    return pl.pallas_call(
        _tail_body,
        grid=(_NTB,),
        in_specs=[pl.BlockSpec((_TB, _E, _R), lambda j: (j, 0, 0))],
        out_specs=[
            pl.BlockSpec((_TB, _E, _R), lambda j: (j, 0, 0)),
            pl.BlockSpec((_TB, _E), lambda j: (j, 0)),
            pl.BlockSpec(memory_space=pltpu.SMEM),
        ],
        out_shape=[
            jax.ShapeDtypeStruct((_B * _S, _E, _R), jnp.float32),
            jax.ShapeDtypeStruct((_B * _S, _E), jnp.float32),
            jax.ShapeDtypeStruct((1, 1), jnp.float32),
        ],
        scratch_shapes=[pltpu.SMEM((1,), jnp.float32)],
    )(routing)


def _router_body(cs_ref, sp_ref, mult_ref, sel_ref):
    sp = sp_ref[0, 0]
    scores = cs_ref[...] * (1.0 + sp)
    iota = jax.lax.broadcasted_iota(jnp.int32, scores.shape, 1)
    v1 = jnp.max(scores, axis=-1, keepdims=True)
    i1 = jnp.min(jnp.where(scores == v1, iota, _E), axis=-1, keepdims=True)
    masked = jnp.where(iota == i1, -jnp.inf, scores)
    v2 = jnp.max(masked, axis=-1, keepdims=True)
    i2 = jnp.min(jnp.where(masked == v2, iota, _E), axis=-1, keepdims=True)
    e2 = jnp.exp(v2 - v1)
    denom = 1.0 + e2
    mult_ref[...] = jnp.concatenate([1.0 / denom, e2 / denom], axis=-1)
    sel_ref[...] = jnp.concatenate([i1, i2], axis=-1)


def _router_call(cs_flat, pen):
    return pl.pallas_call(
        _router_body,
        grid=(_NTB,),
        in_specs=[
            pl.BlockSpec((_TB, _E), lambda j: (j, 0)),
            pl.BlockSpec(memory_space=pltpu.SMEM),
        ],
        out_specs=[
            pl.BlockSpec((_TB, 2), lambda j: (j, 0)),
            pl.BlockSpec((_TB, 2), lambda j: (j, 0)),
        ],
        out_shape=[
            jax.ShapeDtypeStruct((_B * _S, 2), jnp.float32),
            jax.ShapeDtypeStruct((_B * _S, 2), jnp.int32),
        ],
    )(cs_flat, pen)


def kernel(x, hn, top_k, W_ih, W_hh):
    xT = jnp.swapaxes(x, 0, 1)
    wih_t = W_ih.T
    whh_t = W_hh.T
    ys, hT = _gru_call(xT, wih_t, whh_t, hn[0])
    out = jnp.swapaxes(ys, 0, 1)
    routing = out.reshape(_B * _S, _E, _R)
    expression, cs_flat, pen = _tail_call(routing)
    multiplier, selected = _router_call(cs_flat, pen)
    hn_out = hT[None]
    speciality_penalty = pen[0, 0]
    cosine_sims_r = cs_flat.reshape(_B, _S, _E)
    tka = jnp.asarray(top_k)
    expression_loss = (tka - tka).astype(x.dtype)
    return (multiplier, selected, expression, hn_out, speciality_penalty,
            cosine_sims_r, expression_loss)


# speedup vs baseline: 1.0431x; 1.0431x over previous
import jax
import jax.numpy as jnp
from jax.experimental import pallas as pl
from jax.experimental.pallas import tpu as pltpu

_B, _S, _I = 4, 2048, 1024
_E, _R = 8, 128
_H = _E * _R
_CHUNK = 256
_NCH = _S // _CHUNK
_UNROLL = 8
_TB = 512
_NTB = (_B * _S) // _TB


def _gru_body(xT_ref, wih_ref, whh_ref, h0_ref, ys_ref, hT_ref, h_scr, gi_scr):
    i = pl.program_id(0)

    @pl.when(i == 0)
    def _init():
        h_scr[...] = h0_ref[...]

    x2 = xT_ref[...].reshape(_CHUNK * _B, _I)
    gi_scr[...] = jnp.dot(x2, wih_ref[...])

    def step2(u, h):
        gi8 = gi_scr[pl.ds(u * _UNROLL * _B, _UNROLL * _B), :]
        for k in range(_UNROLL):
            gi_t = gi8[k * _B:(k + 1) * _B, :]
            gh_rz = jnp.dot(h, whh_ref[:, :2 * _H])
            gh_n = jnp.dot(h, whh_ref[:, 2 * _H:])
            r = jax.nn.sigmoid(gi_t[:, :_H] + gh_rz[:, :_H])
            z = jax.nn.sigmoid(gi_t[:, _H:2 * _H] + gh_rz[:, _H:])
            n = jnp.tanh(gi_t[:, 2 * _H:] + r * gh_n)
            h = (1.0 - z) * n + z * h
            ys_ref[u * _UNROLL + k] = h
        return h

    h_fin = jax.lax.fori_loop(0, _CHUNK // _UNROLL, step2, h_scr[...])
    h_scr[...] = h_fin
    hT_ref[...] = h_fin


def _gru_call(xT, wih_t, whh_t, h0):
    return pl.pallas_call(
        _gru_body,
        grid=(_NCH,),
        in_specs=[
            pl.BlockSpec((_CHUNK, _B, _I), lambda i: (i, 0, 0)),
            pl.BlockSpec((_I, 3 * _H), lambda i: (0, 0)),
            pl.BlockSpec((_H, 3 * _H), lambda i: (0, 0)),
            pl.BlockSpec((_B, _H), lambda i: (0, 0)),
        ],
        out_specs=[
            pl.BlockSpec((_CHUNK, _B, _H), lambda i: (i, 0, 0)),
            pl.BlockSpec((_B, _H), lambda i: (0, 0)),
        ],
        out_shape=[
            jax.ShapeDtypeStruct((_S, _B, _H), jnp.float32),
            jax.ShapeDtypeStruct((_B, _H), jnp.float32),
        ],
        scratch_shapes=[
            pltpu.VMEM((_B, _H), jnp.float32),
            pltpu.VMEM((_CHUNK * _B, 3 * _H), jnp.float32),
        ],
    )(xT, wih_t, whh_t, h0)


def _tail_body(v_ref, rn_ref, cs_ref, pen_ref, acc_ref):
    j = pl.program_id(0)
    v = v_ref[...]
    n = jnp.sqrt(jnp.sum(v * v, axis=-1, keepdims=True))
    rn = v / jnp.maximum(n, 1e-12)
    rn_ref[...] = rn
    s = jnp.sum(rn * rn, axis=-1)
    na = jnp.maximum(jnp.sqrt(s), 1e-8)
    cs_ref[...] = 1.0 - s / (na * na)

    dsq = {}
    for a in range(_E):
        for b in range(a, _E):
            g = jnp.sum(rn[:, a, :] * rn[:, b, :], axis=-1, keepdims=True)
            d = g - (1.0 if a == b else 0.0)
            dsq[(a, b)] = d * d
    pen_tok = jnp.zeros((_TB, 1), jnp.float32)
    for a in range(_E):
        rowsq = jnp.zeros((_TB, 1), jnp.float32)
        for b in range(_E):
            rowsq = rowsq + dsq[(min(a, b), max(a, b))]
        m = jnp.maximum(jnp.sqrt(rowsq), 1e-12)
        pen_tok = pen_tok + rowsq / (m * m)
    blk = jnp.sum(pen_tok)

    @pl.when(j == 0)
    def _first():
        acc_ref[0] = blk

    @pl.when(j > 0)
    def _rest():
        acc_ref[0] = acc_ref[0] + blk

    @pl.when(j == _NTB - 1)
    def _last():
        pen_ref[0, 0] = acc_ref[0] / float(_B * _S)


def _tail_call(routing):
    return pl.pallas_call(
        _tail_body,
        grid=(_NTB,),
        in_specs=[pl.BlockSpec((_TB, _E, _R), lambda j: (j, 0, 0))],
        out_specs=[
            pl.BlockSpec((_TB, _E, _R), lambda j: (j, 0, 0)),
            pl.BlockSpec((_TB, _E), lambda j: (j, 0)),
            pl.BlockSpec(memory_space=pltpu.SMEM),
        ],
        out_shape=[
            jax.ShapeDtypeStruct((_B * _S, _E, _R), jnp.float32),
            jax.ShapeDtypeStruct((_B * _S, _E), jnp.float32),
            jax.ShapeDtypeStruct((1, 1), jnp.float32),
        ],
        scratch_shapes=[pltpu.SMEM((1,), jnp.float32)],
    )(routing)


def _router_body(cs_ref, sp_ref, mult_ref, sel_ref):
    sp = sp_ref[0, 0]
    scores = cs_ref[...] * (1.0 + sp)
    iota = jax.lax.broadcasted_iota(jnp.int32, scores.shape, 1)
    v1 = jnp.max(scores, axis=-1, keepdims=True)
    i1 = jnp.min(jnp.where(scores == v1, iota, _E), axis=-1, keepdims=True)
    masked = jnp.where(iota == i1, -jnp.inf, scores)
    v2 = jnp.max(masked, axis=-1, keepdims=True)
    i2 = jnp.min(jnp.where(masked == v2, iota, _E), axis=-1, keepdims=True)
    e2 = jnp.exp(v2 - v1)
    denom = 1.0 + e2
    mult_ref[...] = jnp.concatenate([1.0 / denom, e2 / denom], axis=-1)
    sel_ref[...] = jnp.concatenate([i1, i2], axis=-1)


def _router_call(cs_flat, pen):
    return pl.pallas_call(
        _router_body,
        grid=(_NTB,),
        in_specs=[
            pl.BlockSpec((_TB, _E), lambda j: (j, 0)),
            pl.BlockSpec(memory_space=pltpu.SMEM),
        ],
        out_specs=[
            pl.BlockSpec((_TB, 2), lambda j: (j, 0)),
            pl.BlockSpec((_TB, 2), lambda j: (j, 0)),
        ],
        out_shape=[
            jax.ShapeDtypeStruct((_B * _S, 2), jnp.float32),
            jax.ShapeDtypeStruct((_B * _S, 2), jnp.int32),
        ],
    )(cs_flat, pen)


def kernel(x, hn, top_k, W_ih, W_hh):
    xT = jnp.swapaxes(x, 0, 1)
    wih_t = W_ih.T
    whh_t = W_hh.T
    ys, hT = _gru_call(xT, wih_t, whh_t, hn[0])
    out = jnp.swapaxes(ys, 0, 1)
    routing = out.reshape(_B * _S, _E, _R)
    expression, cs_flat, pen = _tail_call(routing)
    multiplier, selected = _router_call(cs_flat, pen)
    hn_out = hT[None]
    speciality_penalty = pen[0, 0]
    cosine_sims_r = cs_flat.reshape(_B, _S, _E)
    tka = jnp.asarray(top_k)
    expression_loss = (tka - tka).astype(x.dtype)
    return (multiplier, selected, expression, hn_out, speciality_penalty,
            cosine_sims_r, expression_loss)


# speedup vs baseline: 1.0499x; 1.0065x over previous
import jax
import jax.numpy as jnp
from jax.experimental import pallas as pl
from jax.experimental.pallas import tpu as pltpu

_B, _S, _I = 4, 2048, 1024
_E, _R = 8, 128
_H = _E * _R
_CHUNK = 256
_NCH = _S // _CHUNK
_UNROLL = 16
_TB = 512
_NTB = (_B * _S) // _TB


def _gru_body(xT_ref, wih_ref, whh_ref, h0_ref, ys_ref, hT_ref, h_scr, gi_scr):
    i = pl.program_id(0)

    @pl.when(i == 0)
    def _init():
        h_scr[...] = h0_ref[...]

    x2 = xT_ref[...].reshape(_CHUNK * _B, _I)
    gi_scr[...] = jnp.dot(x2, wih_ref[...])

    def step2(u, h):
        gi8 = gi_scr[pl.ds(u * _UNROLL * _B, _UNROLL * _B), :]
        for k in range(_UNROLL):
            gi_t = gi8[k * _B:(k + 1) * _B, :]
            gh_rz = jnp.dot(h, whh_ref[:, :2 * _H])
            gh_n = jnp.dot(h, whh_ref[:, 2 * _H:])
            r = jax.nn.sigmoid(gi_t[:, :_H] + gh_rz[:, :_H])
            z = jax.nn.sigmoid(gi_t[:, _H:2 * _H] + gh_rz[:, _H:])
            n = jnp.tanh(gi_t[:, 2 * _H:] + r * gh_n)
            h = (1.0 - z) * n + z * h
            ys_ref[u * _UNROLL + k] = h
        return h

    h_fin = jax.lax.fori_loop(0, _CHUNK // _UNROLL, step2, h_scr[...])
    h_scr[...] = h_fin
    hT_ref[...] = h_fin


def _gru_call(xT, wih_t, whh_t, h0):
    return pl.pallas_call(
        _gru_body,
        grid=(_NCH,),
        in_specs=[
            pl.BlockSpec((_CHUNK, _B, _I), lambda i: (i, 0, 0)),
            pl.BlockSpec((_I, 3 * _H), lambda i: (0, 0)),
            pl.BlockSpec((_H, 3 * _H), lambda i: (0, 0)),
            pl.BlockSpec((_B, _H), lambda i: (0, 0)),
        ],
        out_specs=[
            pl.BlockSpec((_CHUNK, _B, _H), lambda i: (i, 0, 0)),
            pl.BlockSpec((_B, _H), lambda i: (0, 0)),
        ],
        out_shape=[
            jax.ShapeDtypeStruct((_S, _B, _H), jnp.float32),
            jax.ShapeDtypeStruct((_B, _H), jnp.float32),
        ],
        scratch_shapes=[
            pltpu.VMEM((_B, _H), jnp.float32),
            pltpu.VMEM((_CHUNK * _B, 3 * _H), jnp.float32),
        ],
    )(xT, wih_t, whh_t, h0)


def _tail_body(v_ref, rn_ref, cs_ref, pen_ref, acc_ref):
    j = pl.program_id(0)
    v = v_ref[...]
    n = jnp.sqrt(jnp.sum(v * v, axis=-1, keepdims=True))
    rn = v / jnp.maximum(n, 1e-12)
    rn_ref[...] = rn
    s = jnp.sum(rn * rn, axis=-1)
    na = jnp.maximum(jnp.sqrt(s), 1e-8)
    cs_ref[...] = 1.0 - s / (na * na)

    dsq = {}
    for a in range(_E):
        for b in range(a, _E):
            g = jnp.sum(rn[:, a, :] * rn[:, b, :], axis=-1, keepdims=True)
            d = g - (1.0 if a == b else 0.0)
            dsq[(a, b)] = d * d
    pen_tok = jnp.zeros((_TB, 1), jnp.float32)
    for a in range(_E):
        rowsq = jnp.zeros((_TB, 1), jnp.float32)
        for b in range(_E):
            rowsq = rowsq + dsq[(min(a, b), max(a, b))]
        m = jnp.maximum(jnp.sqrt(rowsq), 1e-12)
        pen_tok = pen_tok + rowsq / (m * m)
    blk = jnp.sum(pen_tok)

    @pl.when(j == 0)
    def _first():
        acc_ref[0] = blk

    @pl.when(j > 0)
    def _rest():
        acc_ref[0] = acc_ref[0] + blk

    @pl.when(j == _NTB - 1)
    def _last():
        pen_ref[0, 0] = acc_ref[0] / float(_B * _S)


def _tail_call(routing):
    return pl.pallas_call(
        _tail_body,
        grid=(_NTB,),
        in_specs=[pl.BlockSpec((_TB, _E, _R), lambda j: (j, 0, 0))],
        out_specs=[
            pl.BlockSpec((_TB, _E, _R), lambda j: (j, 0, 0)),
            pl.BlockSpec((_TB, _E), lambda j: (j, 0)),
            pl.BlockSpec(memory_space=pltpu.SMEM),
        ],
        out_shape=[
            jax.ShapeDtypeStruct((_B * _S, _E, _R), jnp.float32),
            jax.ShapeDtypeStruct((_B * _S, _E), jnp.float32),
            jax.ShapeDtypeStruct((1, 1), jnp.float32),
        ],
        scratch_shapes=[pltpu.SMEM((1,), jnp.float32)],
    )(routing)


def _router_body(cs_ref, sp_ref, mult_ref, sel_ref):
    sp = sp_ref[0, 0]
    scores = cs_ref[...] * (1.0 + sp)
    iota = jax.lax.broadcasted_iota(jnp.int32, scores.shape, 1)
    v1 = jnp.max(scores, axis=-1, keepdims=True)
    i1 = jnp.min(jnp.where(scores == v1, iota, _E), axis=-1, keepdims=True)
    masked = jnp.where(iota == i1, -jnp.inf, scores)
    v2 = jnp.max(masked, axis=-1, keepdims=True)
    i2 = jnp.min(jnp.where(masked == v2, iota, _E), axis=-1, keepdims=True)
    e2 = jnp.exp(v2 - v1)
    denom = 1.0 + e2
    mult_ref[...] = jnp.concatenate([1.0 / denom, e2 / denom], axis=-1)
    sel_ref[...] = jnp.concatenate([i1, i2], axis=-1)


def _router_call(cs_flat, pen):
    return pl.pallas_call(
        _router_body,
        grid=(_NTB,),
        in_specs=[
            pl.BlockSpec((_TB, _E), lambda j: (j, 0)),
            pl.BlockSpec(memory_space=pltpu.SMEM),
        ],
        out_specs=[
            pl.BlockSpec((_TB, 2), lambda j: (j, 0)),
            pl.BlockSpec((_TB, 2), lambda j: (j, 0)),
        ],
        out_shape=[
            jax.ShapeDtypeStruct((_B * _S, 2), jnp.float32),
            jax.ShapeDtypeStruct((_B * _S, 2), jnp.int32),
        ],
    )(cs_flat, pen)


def kernel(x, hn, top_k, W_ih, W_hh):
    xT = jnp.swapaxes(x, 0, 1)
    wih_t = W_ih.T
    whh_t = W_hh.T
    ys, hT = _gru_call(xT, wih_t, whh_t, hn[0])
    out = jnp.swapaxes(ys, 0, 1)
    routing = out.reshape(_B * _S, _E, _R)
    expression, cs_flat, pen = _tail_call(routing)
    multiplier, selected = _router_call(cs_flat, pen)
    hn_out = hT[None]
    speciality_penalty = pen[0, 0]
    cosine_sims_r = cs_flat.reshape(_B, _S, _E)
    tka = jnp.asarray(top_k)
    expression_loss = (tka - tka).astype(x.dtype)
    return (multiplier, selected, expression, hn_out, speciality_penalty,
            cosine_sims_r, expression_loss)


# speedup vs baseline: 1.0577x; 1.0074x over previous
import functools

import jax
import jax.numpy as jnp
from jax.experimental import pallas as pl
from jax.experimental.pallas import tpu as pltpu
from jax.experimental.pallas import tpu_sc as plsc

_B, _S, _I = 4, 2048, 1024
_E, _R = 8, 128
_H = _E * _R
_CHUNK = 256
_NCH = _S // _CHUNK
_UNROLL = 16
_TB = 512
_NTB = (_B * _S) // _TB


def _gru_body(xT_ref, wih_ref, whh_ref, h0_ref, ys_ref, hT_ref, h_scr, gi_scr):
    i = pl.program_id(0)

    @pl.when(i == 0)
    def _init():
        h_scr[...] = h0_ref[...]

    x2 = xT_ref[...].reshape(_CHUNK * _B, _I)
    gi_scr[...] = jnp.dot(x2, wih_ref[...])

    def step2(u, h):
        gi8 = gi_scr[pl.ds(u * _UNROLL * _B, _UNROLL * _B), :]
        for k in range(_UNROLL):
            gi_t = gi8[k * _B:(k + 1) * _B, :]
            gh_rz = jnp.dot(h, whh_ref[:, :2 * _H])
            gh_n = jnp.dot(h, whh_ref[:, 2 * _H:])
            r = jax.nn.sigmoid(gi_t[:, :_H] + gh_rz[:, :_H])
            z = jax.nn.sigmoid(gi_t[:, _H:2 * _H] + gh_rz[:, _H:])
            n = jnp.tanh(gi_t[:, 2 * _H:] + r * gh_n)
            h = (1.0 - z) * n + z * h
            ys_ref[u * _UNROLL + k] = h
        return h

    h_fin = jax.lax.fori_loop(0, _CHUNK // _UNROLL, step2, h_scr[...])
    h_scr[...] = h_fin
    hT_ref[...] = h_fin


def _gru_call(xT, wih_t, whh_t, h0):
    return pl.pallas_call(
        _gru_body,
        grid=(_NCH,),
        in_specs=[
            pl.BlockSpec((_CHUNK, _B, _I), lambda i: (i, 0, 0)),
            pl.BlockSpec((_I, 3 * _H), lambda i: (0, 0)),
            pl.BlockSpec((_H, 3 * _H), lambda i: (0, 0)),
            pl.BlockSpec((_B, _H), lambda i: (0, 0)),
        ],
        out_specs=[
            pl.BlockSpec((_CHUNK, _B, _H), lambda i: (i, 0, 0)),
            pl.BlockSpec((_B, _H), lambda i: (0, 0)),
        ],
        out_shape=[
            jax.ShapeDtypeStruct((_S, _B, _H), jnp.float32),
            jax.ShapeDtypeStruct((_B, _H), jnp.float32),
        ],
        scratch_shapes=[
            pltpu.VMEM((_B, _H), jnp.float32),
            pltpu.VMEM((_CHUNK * _B, 3 * _H), jnp.float32),
        ],
    )(xT, wih_t, whh_t, h0)


def _tail_body(v_ref, rn_ref, cs_ref, cst_ref, pen_ref, acc_ref):
    j = pl.program_id(0)
    v = v_ref[...]
    n = jnp.sqrt(jnp.sum(v * v, axis=-1, keepdims=True))
    rn = v / jnp.maximum(n, 1e-12)
    rn_ref[...] = rn
    s = jnp.sum(rn * rn, axis=-1)
    na = jnp.maximum(jnp.sqrt(s), 1e-8)
    cs = 1.0 - s / (na * na)
    cs_ref[...] = cs
    cst_ref[...] = cs.T

    dsq = {}
    for a in range(_E):
        for b in range(a, _E):
            g = jnp.sum(rn[:, a, :] * rn[:, b, :], axis=-1, keepdims=True)
            d = g - (1.0 if a == b else 0.0)
            dsq[(a, b)] = d * d
    pen_tok = jnp.zeros((_TB, 1), jnp.float32)
    for a in range(_E):
        rowsq = jnp.zeros((_TB, 1), jnp.float32)
        for b in range(_E):
            rowsq = rowsq + dsq[(min(a, b), max(a, b))]
        m = jnp.maximum(jnp.sqrt(rowsq), 1e-12)
        pen_tok = pen_tok + rowsq / (m * m)
    blk = jnp.sum(pen_tok)

    @pl.when(j == 0)
    def _first():
        acc_ref[0] = blk

    @pl.when(j > 0)
    def _rest():
        acc_ref[0] = acc_ref[0] + blk

    @pl.when(j == _NTB - 1)
    def _last():
        pen_ref[0, 0] = acc_ref[0] / float(_B * _S)


def _tail_call(routing):
    return pl.pallas_call(
        _tail_body,
        grid=(_NTB,),
        in_specs=[pl.BlockSpec((_TB, _E, _R), lambda j: (j, 0, 0))],
        out_specs=[
            pl.BlockSpec((_TB, _E, _R), lambda j: (j, 0, 0)),
            pl.BlockSpec((_TB, _E), lambda j: (j, 0)),
            pl.BlockSpec((_E, _TB), lambda j: (0, j)),
            pl.BlockSpec(memory_space=pltpu.SMEM),
        ],
        out_shape=[
            jax.ShapeDtypeStruct((_B * _S, _E, _R), jnp.float32),
            jax.ShapeDtypeStruct((_B * _S, _E), jnp.float32),
            jax.ShapeDtypeStruct((_E, _B * _S), jnp.float32),
            jax.ShapeDtypeStruct((1, 1), jnp.float32),
        ],
        scratch_shapes=[pltpu.SMEM((1,), jnp.float32)],
    )(routing)


def _sc_router_call(cs_t_flat, scale16):
    info = plsc.get_sparse_core_info()
    nc, ns, nl = info.num_cores, info.num_subcores, info.num_lanes
    nw = nc * ns
    tok = _B * _S
    per = tok // nw
    groups = per // nl
    mesh = plsc.VectorSubcoreMesh(core_axis_name="c", subcore_axis_name="s")

    @functools.partial(
        pl.kernel,
        out_type=(jax.ShapeDtypeStruct((2 * tok,), jnp.float32),
                  jax.ShapeDtypeStruct((2 * tok,), jnp.int32)),
        mesh=mesh,
        scratch_types=[
            pltpu.VMEM((_E, per), jnp.float32),
            pltpu.VMEM((2, per), jnp.float32),
            pltpu.VMEM((2, per), jnp.int32),
            pltpu.VMEM((nl,), jnp.float32),
        ],
    )
    def sc_router(cs_hbm, sc_hbm, mult_hbm, sel_hbm, rows, mout, sout, sca):
        wid = jax.lax.axis_index("s") * nc + jax.lax.axis_index("c")
        base = wid * per
        for e in range(_E):
            pltpu.sync_copy(cs_hbm.at[pl.ds(e * tok + base, per)], rows.at[e])
        pltpu.sync_copy(sc_hbm, sca)
        scale = sca[...]
        for g in range(groups):
            sl = pl.ds(g * nl, nl)
            vs = [rows[e, sl] * scale for e in range(_E)]
            v1 = vs[0]
            for e in range(1, _E):
                v1 = jnp.maximum(v1, vs[e])
            i1 = jnp.zeros((nl,), jnp.int32)
            for e in range(_E - 1, -1, -1):
                i1 = jnp.where(vs[e] == v1, jnp.int32(e), i1)
            ms = [jnp.where(i1 == e, -jnp.inf, vs[e]) for e in range(_E)]
            v2 = ms[0]
            for e in range(1, _E):
                v2 = jnp.maximum(v2, ms[e])
            i2 = jnp.zeros((nl,), jnp.int32)
            for e in range(_E - 1, -1, -1):
                i2 = jnp.where(ms[e] == v2, jnp.int32(e), i2)
            e2 = jnp.exp(v2 - v1)
            den = 1.0 + e2
            mout[0, sl] = 1.0 / den
            mout[1, sl] = e2 / den
            sout[0, sl] = i1
            sout[1, sl] = i2
        pltpu.sync_copy(mout.at[0], mult_hbm.at[pl.ds(base, per)])
        pltpu.sync_copy(mout.at[1], mult_hbm.at[pl.ds(tok + base, per)])
        pltpu.sync_copy(sout.at[0], sel_hbm.at[pl.ds(base, per)])
        pltpu.sync_copy(sout.at[1], sel_hbm.at[pl.ds(tok + base, per)])

    return sc_router(cs_t_flat, scale16)


def kernel(x, hn, top_k, W_ih, W_hh):
    xT = jnp.swapaxes(x, 0, 1)
    wih_t = W_ih.T
    whh_t = W_hh.T
    ys, hT = _gru_call(xT, wih_t, whh_t, hn[0])
    out = jnp.swapaxes(ys, 0, 1)
    routing = out.reshape(_B * _S, _E, _R)
    expression, cs_flat, cs_t, pen = _tail_call(routing)
    scale16 = jnp.broadcast_to(1.0 + pen[0, 0], (16,))
    mult_flat, sel_flat = _sc_router_call(cs_t.reshape(_E * _B * _S), scale16)
    multiplier = mult_flat.reshape(2, _B * _S).T
    selected = sel_flat.reshape(2, _B * _S).T
    hn_out = hT[None]
    speciality_penalty = pen[0, 0]
    cosine_sims_r = cs_flat.reshape(_B, _S, _E)
    tka = jnp.asarray(top_k)
    expression_loss = (tka - tka).astype(x.dtype)
    return (multiplier, selected, expression, hn_out, speciality_penalty,
            cosine_sims_r, expression_loss)
